# baseline (speedup 1.0000x reference)
import jax
import jax.numpy as jnp
from jax.experimental import pallas as pl
from jax.experimental.pallas import tpu as pltpu

_B = 8
_N = 2048
_R = 256
_NR = _N // _R


def _diag_body(ns_ref, cov_ref, het_ref, out_ref):
    row = jax.lax.broadcasted_iota(jnp.int32, (_R, _R), 0)
    col = jax.lax.broadcasted_iota(jnp.int32, (_R, _R), 1)
    ens = jnp.exp(ns_ref[0])
    val = jnp.exp(het_ref[...]) + ens
    out_ref[0] = cov_ref[0] + jnp.where(col == row, val, 0.0)


def kernel(cov, embeddings, noise_scale):
    het = embeddings[:, :, -1].reshape(_B * _N, 1)
    grid = (_B, _NR)
    out = pl.pallas_call(
        _diag_body,
        grid=grid,
        in_specs=[
            pl.BlockSpec(memory_space=pltpu.SMEM),
            pl.BlockSpec((1, _R, _R), lambda b, r: (b, r, r)),
            pl.BlockSpec((_R, 1), lambda b, r: (b * _NR + r, 0)),
        ],
        out_specs=pl.BlockSpec((1, _R, _R), lambda b, r: (b, r, r)),
        out_shape=jax.ShapeDtypeStruct((_B, _N, _N), jnp.float32),
        input_output_aliases={1: 0},
    )(noise_scale, cov, het)
    return out
